# Initial kernel scaffold; baseline (speedup 1.0000x reference)
import jax
import jax.numpy as jnp
from jax.experimental import pallas as pl


def kernel(x, edge_index, edge_vals, W):
    raise NotImplementedError("write your pallas kernel here")


# speedup vs baseline: 2.0274x; 2.0274x over previous
import jax
import jax.numpy as jnp
from jax import lax
from jax.experimental import pallas as pl
from jax.experimental.pallas import tpu as pltpu
from jax.experimental.pallas import tpu_sc as plsc

N_NODES = 10000
N_EDGES = 160000
DIM = 256
HALF = DIM // 2

NC = 2
NS = 16
L = 16

CHUNK = 128
CPT = -(-N_EDGES // (NS * CHUNK))
E_PAD = NS * CPT * CHUNK

RCHUNK = 200
NRCH = N_NODES // RCHUNK
RITERS = -(-NRCH // NS)


def _sc_body(x2_hbm, row_hbm, col_hbm, val_hbm, w_hbm, out_hbm,
             acc_sh, colv, rowv, valv, rows, stage, w_v, sem):
    c = lax.axis_index("c")
    s = lax.axis_index("s")
    off = (c * N_NODES).astype(jnp.int32)

    def zero_row(r, _):
        for d in range(HALF // L):
            stage[r, pl.ds(d * L, L)] = jnp.zeros((L,), jnp.float32)
        return 0
    lax.fori_loop(0, RCHUNK, zero_row, 0)
    for t in range(RITERS):
        k = s + NS * t
        @pl.when(k < NRCH)
        def _():
            r0 = pl.multiple_of(k * RCHUNK, RCHUNK)
            pltpu.sync_copy(stage, acc_sh.at[pl.ds(r0, RCHUNK)])

    woff = pl.multiple_of(c * HALF, HALF)
    pltpu.sync_copy(w_hbm.at[pl.ds(woff, HALF)], w_v)
    plsc.subcore_barrier()

    def chunk_body(j, _):
        base = pl.multiple_of((s * CPT + j) * CHUNK, CHUNK)
        pltpu.sync_copy(col_hbm.at[pl.ds(base, CHUNK)], colv)
        pltpu.sync_copy(row_hbm.at[pl.ds(base, CHUNK)], rowv)
        pltpu.sync_copy(val_hbm.at[pl.ds(base * L, CHUNK * L)], valv)
        for g in range(CHUNK // L):
            sl = pl.ds(g * L, L)
            colv[sl] = colv[sl] + off
        pltpu.async_copy(x2_hbm.at[colv], rows, sem).wait()
        def edge_body(e, _):
            sv = valv[pl.ds(pl.multiple_of(e * L, L), L)]
            for d in range(HALF // L):
                sl = pl.ds(d * L, L)
                rows[e, sl] = rows[e, sl] * sv
            return 0
        lax.fori_loop(0, CHUNK, edge_body, 0)
        pltpu.sync_copy(rows, acc_sh.at[rowv], add=True)
        return 0
    lax.fori_loop(0, CPT, chunk_body, 0)
    plsc.subcore_barrier()

    for t in range(RITERS):
        k = s + NS * t
        @pl.when(k < NRCH)
        def _():
            r0 = pl.multiple_of(k * RCHUNK, RCHUNK)
            pltpu.sync_copy(acc_sh.at[pl.ds(r0, RCHUNK)], stage)
            def relu_row(r, _):
                for d in range(HALF // L):
                    sl = pl.ds(d * L, L)
                    stage[r, sl] = jnp.maximum(stage[r, sl] * w_v[sl], 0.0)
                return 0
            lax.fori_loop(0, RCHUNK, relu_row, 0)
            pltpu.sync_copy(stage,
                            out_hbm.at[pl.ds(r0, RCHUNK), pl.ds(woff, HALF)])


def _make_kernel():
    mesh = plsc.VectorSubcoreMesh(core_axis_name="c", subcore_axis_name="s")
    return pl.kernel(
        _sc_body,
        out_type=jax.ShapeDtypeStruct((N_NODES, DIM), jnp.float32),
        mesh=mesh,
        scratch_types=[
            pltpu.VMEM_SHARED((N_NODES, HALF), jnp.float32),
            pltpu.VMEM((CHUNK,), jnp.int32),
            pltpu.VMEM((CHUNK,), jnp.int32),
            pltpu.VMEM((CHUNK * L,), jnp.float32),
            pltpu.VMEM((CHUNK, HALF), jnp.float32),
            pltpu.VMEM((RCHUNK, HALF), jnp.float32),
            pltpu.VMEM((HALF,), jnp.float32),
            pltpu.SemaphoreType.DMA,
        ],
    )


@jax.jit
def kernel(x, edge_index, edge_vals, W):
    row = edge_index[0].astype(jnp.int32)
    col = edge_index[1].astype(jnp.int32)
    pad = E_PAD - N_EDGES
    row1 = jnp.pad(row, (0, pad))
    col1 = jnp.pad(col, (0, pad))
    val1 = jnp.repeat(jnp.pad(edge_vals.astype(jnp.float32), (0, pad)), L)
    x2 = jnp.concatenate([x[:, :HALF], x[:, HALF:]], axis=0)
    w1 = W.reshape(DIM)
    return _make_kernel()(x2, row1, col1, val1, w1)


# speedup vs baseline: 2.4445x; 1.2057x over previous
import jax
import jax.numpy as jnp
from jax import lax
from jax.experimental import pallas as pl
from jax.experimental.pallas import tpu as pltpu
from jax.experimental.pallas import tpu_sc as plsc

N_NODES = 10000
N_EDGES = 160000
DIM = 256
HALF = DIM // 2

NC = 2
NS = 16
L = 16

CHUNK = 64
NBUF = 4
CPT = 160
E_PAD = NS * CPT * CHUNK

RCHUNK = 64
NRCH = N_NODES // RCHUNK
RTAIL = N_NODES - NRCH * RCHUNK
RITERS = -(-NRCH // NS)


def _sc_body(x2_hbm, row_hbm, col_hbm, val_hbm, w_hbm, out_hbm,
             acc_sh, w_v, colv, rowv, valv, rows, gsems, ssems, isems, vsems):
    c = lax.axis_index("c")
    s = lax.axis_index("s")
    off = (c * N_NODES).astype(jnp.int32)

    stage = rows[0]
    def zero_row(r, _):
        for d in range(HALF // L):
            stage[r, pl.ds(d * L, L)] = jnp.zeros((L,), jnp.float32)
        return 0
    lax.fori_loop(0, RCHUNK, zero_row, 0)
    for t in range(RITERS):
        k = s + NS * t
        @pl.when(k < NRCH)
        def _():
            r0 = pl.multiple_of(k * RCHUNK, RCHUNK)
            pltpu.sync_copy(stage, acc_sh.at[pl.ds(r0, RCHUNK)])
    @pl.when(s == 0)
    def _():
        pltpu.sync_copy(stage.at[pl.ds(0, RTAIL)],
                        acc_sh.at[pl.ds(NRCH * RCHUNK, RTAIL)])

    woff = pl.multiple_of(c * HALF, HALF)
    pltpu.sync_copy(w_hbm.at[pl.ds(woff, HALF)], w_v)
    plsc.subcore_barrier()

    ibase = s * (CPT * CHUNK)
    vbase = s * (CPT * CHUNK * L)

    def fire_meta(j, b):
        io = pl.multiple_of(ibase + j * CHUNK, CHUNK)
        vo = pl.multiple_of(vbase + j * (CHUNK * L), CHUNK * L)
        pltpu.async_copy(col_hbm.at[pl.ds(io, CHUNK)], colv[b], isems[b])
        pltpu.async_copy(row_hbm.at[pl.ds(io, CHUNK)], rowv[b], isems[b])
        pltpu.async_copy(val_hbm.at[pl.ds(vo, CHUNK * L)], valv[b], vsems[b])

    def wait_meta_idx(b):
        pltpu.make_async_copy(col_hbm.at[pl.ds(0, CHUNK)], colv[b],
                              isems[b]).wait()
        pltpu.make_async_copy(row_hbm.at[pl.ds(0, CHUNK)], rowv[b],
                              isems[b]).wait()

    def offset_cols(b):
        for g in range(CHUNK // L):
            sl = pl.ds(g * L, L)
            colv[b][sl] = colv[b][sl] + off

    def fire_gather(b):
        pltpu.async_copy(x2_hbm.at[colv[b]], rows[b], gsems[b])

    fire_meta(0, 0)
    fire_meta(1, 1)
    wait_meta_idx(0)
    offset_cols(0)
    fire_gather(0)

    def chunk_body(j, _):
        for b0 in range(NBUF):
            jb = j * NBUF + b0
            b = b0
            bn = (b0 + 1) % NBUF
            b2 = (b0 + 2) % NBUF
            pltpu.make_async_copy(x2_hbm.at[colv[b]], rows[b], gsems[b]).wait()
            pltpu.make_async_copy(val_hbm.at[pl.ds(0, CHUNK * L)], valv[b],
                                  vsems[b]).wait()
            @pl.when(jb + 1 < CPT)
            def _():
                wait_meta_idx(bn)
                offset_cols(bn)
                fire_gather(bn)
            def edge_body(e, _):
                sv = valv[b][pl.ds(pl.multiple_of(e * L, L), L)]
                for d in range(HALF // L):
                    sl = pl.ds(d * L, L)
                    rows[b][e, sl] = rows[b][e, sl] * sv
                return 0
            lax.fori_loop(0, CHUNK, edge_body, 0, unroll=4)
            @pl.when(jb < CPT - 2)
            def _():
                pltpu.async_copy(rows[b], acc_sh.at[rowv[b]], ssems[b],
                                 add=True)
            @pl.when(jb >= CPT - 2)
            def _():
                pltpu.sync_copy(rows[b], acc_sh.at[rowv[b]], add=True)
            @pl.when(jb >= 2)
            def _():
                pltpu.make_async_copy(rows[b2], acc_sh.at[rowv[b2]],
                                      ssems[b2]).wait()
            @pl.when(jb + 2 < CPT)
            def _():
                fire_meta(jb + 2, b2)
        return 0
    lax.fori_loop(0, CPT // NBUF, chunk_body, 0)
    plsc.subcore_barrier()

    def relu_rows(n):
        def relu_row(r, _):
            for d in range(HALF // L):
                sl = pl.ds(d * L, L)
                stage[r, sl] = jnp.maximum(stage[r, sl] * w_v[sl], 0.0)
            return 0
        lax.fori_loop(0, n, relu_row, 0, unroll=2)

    for t in range(RITERS):
        k = s + NS * t
        @pl.when(k < NRCH)
        def _():
            r0 = pl.multiple_of(k * RCHUNK, RCHUNK)
            pltpu.sync_copy(acc_sh.at[pl.ds(r0, RCHUNK)], stage)
            relu_rows(RCHUNK)
            pltpu.sync_copy(stage,
                            out_hbm.at[pl.ds(r0, RCHUNK), pl.ds(woff, HALF)])
    @pl.when(s == 0)
    def _():
        r0 = NRCH * RCHUNK
        pltpu.sync_copy(acc_sh.at[pl.ds(r0, RTAIL)], stage.at[pl.ds(0, RTAIL)])
        relu_rows(RTAIL)
        pltpu.sync_copy(stage.at[pl.ds(0, RTAIL)],
                        out_hbm.at[pl.ds(r0, RTAIL), pl.ds(woff, HALF)])


def _make_kernel():
    mesh = plsc.VectorSubcoreMesh(core_axis_name="c", subcore_axis_name="s")

    def body(x2_hbm, row_hbm, col_hbm, val_hbm, w_hbm, out_hbm, acc_sh, w_v,
             *rest):
        colv = rest[0:NBUF]
        rowv = rest[NBUF:2 * NBUF]
        valv = rest[2 * NBUF:3 * NBUF]
        rows = rest[3 * NBUF:4 * NBUF]
        sems = rest[4 * NBUF:]
        _sc_body(x2_hbm, row_hbm, col_hbm, val_hbm, w_hbm, out_hbm,
                 acc_sh, w_v, colv, rowv, valv, rows,
                 sems[0:NBUF], sems[NBUF:2 * NBUF],
                 sems[2 * NBUF:3 * NBUF], sems[3 * NBUF:4 * NBUF])

    return pl.kernel(
        body,
        out_type=jax.ShapeDtypeStruct((N_NODES, DIM), jnp.float32),
        mesh=mesh,
        scratch_types=[
            pltpu.VMEM_SHARED((N_NODES, HALF), jnp.float32),
            pltpu.VMEM((HALF,), jnp.float32),
        ] + [pltpu.VMEM((CHUNK,), jnp.int32)] * NBUF
          + [pltpu.VMEM((CHUNK,), jnp.int32)] * NBUF
          + [pltpu.VMEM((CHUNK * L,), jnp.float32)] * NBUF
          + [pltpu.VMEM((CHUNK, HALF), jnp.float32)] * NBUF
          + [pltpu.SemaphoreType.DMA] * (4 * NBUF),
    )


@jax.jit
def kernel(x, edge_index, edge_vals, W):
    row = edge_index[0].astype(jnp.int32)
    col = edge_index[1].astype(jnp.int32)
    pad = E_PAD - N_EDGES
    row1 = jnp.pad(row, (0, pad))
    col1 = jnp.pad(col, (0, pad))
    val1 = jnp.repeat(jnp.pad(edge_vals.astype(jnp.float32), (0, pad)), L)
    x2 = jnp.concatenate([x[:, :HALF], x[:, HALF:]], axis=0)
    w1 = W.reshape(DIM)
    return _make_kernel()(x2, row1, col1, val1, w1)


# speedup vs baseline: 3.6449x; 1.4910x over previous
import jax
import jax.numpy as jnp
from jax import lax
from jax.experimental import pallas as pl
from jax.experimental.pallas import tpu as pltpu
from jax.experimental.pallas import tpu_sc as plsc

N_NODES = 10000
N_EDGES = 160000
DIM = 256
HALF = DIM // 2

NC = 2
NS = 16
L = 16

CHUNK = 64
NBUF = 3
CPT = 159
EPT = CPT * CHUNK
E_PAD = NS * EPT
PACK_SHIFT = 14

RCHUNK = 64
NRCH = N_NODES // RCHUNK
RTAIL = N_NODES - NRCH * RCHUNK
RITERS = -(-NRCH // NS)


def _sc_body(x2_hbm, meta_hbm, val_hbm, w_hbm, out_hbm,
             acc_sh, w_v, meta_all, val_all, colv, rowv, rows, gsems, ssems):
    c = lax.axis_index("c")
    s = lax.axis_index("s")

    stage = rows[0]
    def zero_row(r, _):
        for d in range(HALF // L):
            stage[r, pl.ds(d * L, L)] = jnp.zeros((L,), jnp.float32)
        return 0
    lax.fori_loop(0, RCHUNK, zero_row, 0)
    for t in range(RITERS):
        k = s + NS * t
        @pl.when(k < NRCH)
        def _():
            r0 = pl.multiple_of(k * RCHUNK, RCHUNK)
            pltpu.sync_copy(stage, acc_sh.at[pl.ds(r0, RCHUNK)])
    @pl.when(s == 0)
    def _():
        pltpu.sync_copy(stage.at[pl.ds(0, RTAIL)],
                        acc_sh.at[pl.ds(NRCH * RCHUNK, RTAIL)])

    woff = pl.multiple_of(c * HALF, HALF)
    pltpu.sync_copy(w_hbm.at[pl.ds(woff, HALF)], w_v)
    ebase = pl.multiple_of(s * EPT, CHUNK)
    pltpu.sync_copy(meta_hbm.at[pl.ds(ebase, EPT)], meta_all)
    pltpu.sync_copy(val_hbm.at[pl.ds(ebase, EPT)], val_all)
    plsc.subcore_barrier()

    col_mask = jnp.full((L,), (1 << PACK_SHIFT) - 1, jnp.int32)

    def unpack_meta(j, b):
        for g in range(CHUNK // L):
            sl = pl.ds(pl.multiple_of(j * CHUNK, CHUNK) + g * L, L)
            p = meta_all[sl]
            dst = pl.ds(g * L, L)
            colv[b][dst] = ((p & col_mask) << 1) + c
            rowv[b][dst] = lax.shift_right_logical(p, PACK_SHIFT)

    def fire_gather(b):
        pltpu.async_copy(x2_hbm.at[colv[b]], rows[b], gsems[b])

    unpack_meta(0, 0)
    fire_gather(0)

    def chunk_body(j3, _):
        lane = lax.iota(jnp.int32, L)
        for b in range(NBUF):
            jb = j3 * NBUF + b
            bn = (b + 1) % NBUF
            pltpu.make_async_copy(x2_hbm.at[colv[b]], rows[b], gsems[b]).wait()
            @pl.when(jb >= 2)
            def _():
                pltpu.make_async_copy(rows[bn], acc_sh.at[rowv[bn]],
                                      ssems[bn]).wait()
            @pl.when(jb + 1 < CPT)
            def _():
                unpack_meta(jb + 1, bn)
                fire_gather(bn)
            def group_body(g, _):
                vv = val_all[pl.ds(pl.multiple_of(jb * CHUNK, CHUNK) + g * L, L)]
                for i in range(L):
                    sv = vv.at[lane * 0 + i].get(mode="promise_in_bounds")
                    e = g * L + i
                    for d in range(HALF // L):
                        sl = pl.ds(d * L, L)
                        rows[b][e, sl] = rows[b][e, sl] * sv
                return 0
            lax.fori_loop(0, CHUNK // L, group_body, 0)
            @pl.when(jb < CPT - 2)
            def _():
                pltpu.async_copy(rows[b], acc_sh.at[rowv[b]], ssems[b],
                                 add=True)
            @pl.when(jb >= CPT - 2)
            def _():
                pltpu.sync_copy(rows[b], acc_sh.at[rowv[b]], add=True)
        return 0
    lax.fori_loop(0, CPT // NBUF, chunk_body, 0)
    plsc.subcore_barrier()

    def relu_rows(n):
        def relu_row(r, _):
            for d in range(HALF // L):
                sl = pl.ds(d * L, L)
                stage[r, sl] = jnp.maximum(stage[r, sl] * w_v[sl], 0.0)
            return 0
        lax.fori_loop(0, n, relu_row, 0, unroll=2)

    for t in range(RITERS):
        k = s + NS * t
        @pl.when(k < NRCH)
        def _():
            r0 = pl.multiple_of(k * RCHUNK, RCHUNK)
            pltpu.sync_copy(acc_sh.at[pl.ds(r0, RCHUNK)], stage)
            relu_rows(RCHUNK)
            pltpu.sync_copy(stage,
                            out_hbm.at[pl.ds(r0, RCHUNK), pl.ds(woff, HALF)])
    @pl.when(s == 0)
    def _():
        r0 = NRCH * RCHUNK
        pltpu.sync_copy(acc_sh.at[pl.ds(r0, RTAIL)], stage.at[pl.ds(0, RTAIL)])
        relu_rows(RTAIL)
        pltpu.sync_copy(stage.at[pl.ds(0, RTAIL)],
                        out_hbm.at[pl.ds(r0, RTAIL), pl.ds(woff, HALF)])


def _make_kernel():
    mesh = plsc.VectorSubcoreMesh(core_axis_name="c", subcore_axis_name="s")

    def body(x2_hbm, meta_hbm, val_hbm, w_hbm, out_hbm, acc_sh, w_v,
             meta_all, val_all, *rest):
        colv = rest[0:NBUF]
        rowv = rest[NBUF:2 * NBUF]
        rows = rest[2 * NBUF:3 * NBUF]
        sems = rest[3 * NBUF:]
        _sc_body(x2_hbm, meta_hbm, val_hbm, w_hbm, out_hbm,
                 acc_sh, w_v, meta_all, val_all, colv, rowv, rows,
                 sems[0:NBUF], sems[NBUF:2 * NBUF])

    return pl.kernel(
        body,
        out_type=jax.ShapeDtypeStruct((N_NODES, DIM), jnp.float32),
        mesh=mesh,
        scratch_types=[
            pltpu.VMEM_SHARED((N_NODES, HALF), jnp.float32),
            pltpu.VMEM((HALF,), jnp.float32),
            pltpu.VMEM((EPT,), jnp.int32),
            pltpu.VMEM((EPT,), jnp.float32),
        ] + [pltpu.VMEM((CHUNK,), jnp.int32)] * NBUF
          + [pltpu.VMEM((CHUNK,), jnp.int32)] * NBUF
          + [pltpu.VMEM((CHUNK, HALF), jnp.float32)] * NBUF
          + [pltpu.SemaphoreType.DMA] * (2 * NBUF),
    )


@jax.jit
def kernel(x, edge_index, edge_vals, W):
    row = edge_index[0].astype(jnp.int32)
    col = edge_index[1].astype(jnp.int32)
    pad = E_PAD - N_EDGES
    meta = jnp.pad((row << PACK_SHIFT) | col, (0, pad))
    val1 = jnp.pad(edge_vals.astype(jnp.float32), (0, pad))
    x2 = x.reshape(2 * N_NODES, HALF)
    w1 = W.reshape(DIM)
    return _make_kernel()(x2, meta, val1, w1)
